# baseline (speedup 1.0000x reference)
import jax
import jax.numpy as jnp
from jax import lax
from jax.experimental import pallas as pl
from jax.experimental.pallas import tpu as pltpu
from jax.experimental.pallas import tpu_sc as plsc

RES_ = 512
FDIM_ = 32
NPTS = 1000000
OUTD = 3 + FDIM_
QROW = 4 * FDIM_

NC = 2
NS = 16
NW = NC * NS
LANES = 16

B = 80
NV = B // LANES
NPL = 3
KCH = 5
BP = KCH * B
NBATCH = NPTS // BP
NBW = NBATCH // NW
NBR = NBATCH % NW


def _quad_body(a_ref, b_ref, t_ref):
  y = pl.program_id(1)
  a = a_ref[0, :, y % 8, :]
  b = b_ref[0, :, jnp.minimum(y + 1, RES_ - 1) % 8, :]
  eye = jnp.eye(FDIM_, dtype=jnp.float32)
  dn = (((0,), (0,)), ((), ()))
  at = lax.dot_general(a, eye, dn, precision=lax.Precision.HIGHEST)
  bt = lax.dot_general(b, eye, dn, precision=lax.Precision.HIGHEST)
  a1 = jnp.concatenate([at[1:], at[RES_ - 1:]], axis=0)
  b1 = jnp.concatenate([bt[1:], bt[RES_ - 1:]], axis=0)
  t_ref[...] = jnp.concatenate([at, a1, bt, b1], axis=1)


def _build_quad_table(tgrid):
  g4 = tgrid.reshape(3, FDIM_, RES_, RES_)
  line = pl.BlockSpec((1, FDIM_, 8, RES_), lambda p, y: (p, 0, y // 8, 0))
  line_n = pl.BlockSpec((1, FDIM_, 8, RES_),
                        lambda p, y: (p, 0, jnp.minimum(y + 1, RES_ - 1) // 8, 0))
  return pl.pallas_call(
      _quad_body,
      grid=(3, RES_),
      in_specs=[line, line_n],
      out_specs=pl.BlockSpec((RES_, QROW), lambda p, y: (p * RES_ + y, 0)),
      out_shape=jax.ShapeDtypeStruct((3 * RES_ * RES_, QROW), jnp.float32),
  )(g4, g4)


def _tec_body(x_hbm, tab_hbm, out_hbm, xv, idxv, wv, rows, outv, sem, osem):
  cid = lax.axis_index("c")
  sid = lax.axis_index("s")
  wid = sid * NC + cid
  nb = NBW + jnp.where(wid < NBR, 1, 0)
  b0 = wid * NBW + jnp.minimum(wid, NBR)
  iot = lax.iota(jnp.int32, LANES)

  def batch_body(g, _):
    pt0 = (b0 + g) * BP
    xcps = [pltpu.async_copy(x_hbm.at[pl.ds(c * NPTS + pt0, BP)],
                             xv.at[pl.ds(c * BP, BP)], sem) for c in range(3)]
    for cp in xcps:
      cp.wait()

    def chunk_body(k, _):
      kb = k * B

      def idx_body(i, _):
        b = i * LANES
        xc = [xv[pl.ds(c * BP + kb + b, LANES)] for c in range(3)]
        for p, (ca, cb) in enumerate(((0, 1), (1, 2), (2, 0))):
          xf = (xc[ca] + 1.0) * 0.5 * (RES_ - 1)
          yf = (xc[cb] + 1.0) * 0.5 * (RES_ - 1)
          x0 = xf.astype(jnp.int32)
          y0 = yf.astype(jnp.int32)
          fx = xf - x0.astype(jnp.float32)
          fy = yf - y0.astype(jnp.float32)
          gx = 1.0 - fx
          gy = 1.0 - fy
          idxv[p, pl.ds(b, LANES)] = y0 * RES_ + x0 + (p * RES_ * RES_)
          for q, tw in enumerate((gy * gx, gy * fx, fy * gx, fy * fx)):
            wv[p * 4 + q, pl.ds(b, LANES)] = tw
        return 0

      lax.fori_loop(0, NV, idx_body, 0)

      cps = [pltpu.async_copy(tab_hbm.at[idxv.at[p]],
                              rows.at[pl.ds(p * B, B), :], sem)
             for p in range(NPL)]
      for cp in cps:
        cp.wait()

      def acc_body(i, _):
        b = i * LANES
        bi = iot + b
        for c in range(3):
          outv[pl.ds(c * BP + kb + b, LANES)] = xv[pl.ds(c * BP + kb + b, LANES)]
        ws = [wv[t, pl.ds(b, LANES)] for t in range(4 * NPL)]
        rvs = [bi + p * B for p in range(NPL)]
        for ch in range(FDIM_):
          acc = None
          for p in range(NPL):
            for q in range(4):
              cols = jnp.full((LANES,), q * FDIM_ + ch, jnp.int32)
              v = plsc.load_gather(rows, [rvs[p], cols]) * ws[p * 4 + q]
              acc = v if acc is None else acc + v
          outv[pl.ds((3 + ch) * BP + kb + b, LANES)] = acc
        return 0

      lax.fori_loop(0, NV, acc_body, 0)
      return 0

    lax.fori_loop(0, KCH, chunk_body, 0)

    ocps = [pltpu.async_copy(outv.at[pl.ds(r * BP, BP)],
                             out_hbm.at[pl.ds(r * NPTS + pt0, BP)], osem)
            for r in range(OUTD)]
    for cp in ocps:
      cp.wait()
    return 0

  lax.fori_loop(0, nb, batch_body, 0)


@jax.jit
def kernel(x, tgrid):
  tab = _build_quad_table(tgrid)
  mesh = plsc.VectorSubcoreMesh(core_axis_name="c", subcore_axis_name="s",
                                num_cores=NC, num_subcores=NS)
  run = pl.kernel(
      _tec_body,
      out_type=jax.ShapeDtypeStruct((OUTD * NPTS,), jnp.float32),
      mesh=mesh,
      compiler_params=pltpu.CompilerParams(needs_layout_passes=False,
                                           use_tc_tiling_on_sc=False),
      scratch_types=[
          pltpu.VMEM((3 * BP,), jnp.float32),
          pltpu.VMEM((NPL, B), jnp.int32),
          pltpu.VMEM((4 * NPL, B), jnp.float32),
          pltpu.VMEM((NPL * B, QROW), jnp.float32),
          pltpu.VMEM((OUTD * BP,), jnp.float32),
          pltpu.SemaphoreType.DMA,
          pltpu.SemaphoreType.DMA,
      ],
  )
  out_flat = run(x.T.reshape(NPTS * 3), tab)
  return out_flat.reshape(OUTD, NPTS).T


# speedup vs baseline: 1.0404x; 1.0404x over previous
import jax
import jax.numpy as jnp
from jax import lax
from jax.experimental import pallas as pl
from jax.experimental.pallas import tpu as pltpu
from jax.experimental.pallas import tpu_sc as plsc

RES_ = 512
FDIM_ = 32
NPTS = 1000000
OUTD = 3 + FDIM_
QROW = 4 * FDIM_

NC = 2
NS = 16
NW = NC * NS
LANES = 16

B = 80
NV = B // LANES
NPL = 3
KCH = 5
BP = KCH * B
NBATCH = NPTS // BP
NBW = NBATCH // NW
NBR = NBATCH % NW
NSLOT = 2


def _quad_body(a_ref, b_ref, t_ref):
  y = pl.program_id(1)
  a = a_ref[0, :, y % 8, :]
  b = b_ref[0, :, jnp.minimum(y + 1, RES_ - 1) % 8, :]
  eye = jnp.eye(FDIM_, dtype=jnp.float32)
  dn = (((0,), (0,)), ((), ()))
  at = lax.dot_general(a, eye, dn, precision=lax.Precision.HIGHEST)
  bt = lax.dot_general(b, eye, dn, precision=lax.Precision.HIGHEST)
  a1 = jnp.concatenate([at[1:], at[RES_ - 1:]], axis=0)
  b1 = jnp.concatenate([bt[1:], bt[RES_ - 1:]], axis=0)
  t_ref[...] = jnp.concatenate([at, a1, bt, b1], axis=1)


def _build_quad_table(tgrid):
  g4 = tgrid.reshape(3, FDIM_, RES_, RES_)
  line = pl.BlockSpec((1, FDIM_, 8, RES_), lambda p, y: (p, 0, y // 8, 0))
  line_n = pl.BlockSpec((1, FDIM_, 8, RES_),
                        lambda p, y: (p, 0, jnp.minimum(y + 1, RES_ - 1) // 8, 0))
  return pl.pallas_call(
      _quad_body,
      grid=(3, RES_),
      in_specs=[line, line_n],
      out_specs=pl.BlockSpec((RES_, QROW), lambda p, y: (p * RES_ + y, 0)),
      out_shape=jax.ShapeDtypeStruct((3 * RES_ * RES_, QROW), jnp.float32),
  )(g4, g4)


def _tec_body(x_hbm, tab_hbm, out_hbm, xv, idxv, wv, rows, outv,
              gsem0, gsem1, osem, xsem):
  cid = lax.axis_index("c")
  sid = lax.axis_index("s")
  wid = sid * NC + cid
  nb = NBW + jnp.where(wid < NBR, 1, 0)
  b0 = wid * NBW + jnp.minimum(wid, NBR)
  iot = lax.iota(jnp.int32, LANES)

  def gfire(k, slot, sem):
    for p in range(NPL):
      pltpu.async_copy(tab_hbm.at[idxv.at[k * NPL + p]],
                       rows.at[pl.ds((slot * NPL + p) * B, B), :], sem)

  def gdrain(slot, sem):
    for p in range(NPL):
      pltpu.make_async_copy(tab_hbm.at[idxv.at[p]],
                            rows.at[pl.ds((slot * NPL + p) * B, B), :],
                            sem).wait()

  def odrain(pt0):
    for r in range(OUTD):
      pltpu.make_async_copy(outv.at[pl.ds(r * BP, BP)],
                            out_hbm.at[pl.ds(r * NPTS + pt0, BP)], osem).wait()

  def batch_body(g, _):
    pt0 = (b0 + g) * BP
    sb = (g % 2) * (OUTD * BP)

    @pl.when(g >= 2)
    def _():
      odrain(pt0)

    for c in range(3):
      pltpu.async_copy(x_hbm.at[pl.ds(c * NPTS + pt0, BP)],
                       xv.at[pl.ds(c * BP, BP)], xsem)
    for c in range(3):
      pltpu.make_async_copy(x_hbm.at[pl.ds(c * NPTS + pt0, BP)],
                            xv.at[pl.ds(c * BP, BP)], xsem).wait()

    def idx_body(i, _):
      k = i // NV
      b = (i % NV) * LANES
      xc = [xv[pl.ds(c * BP + i * LANES, LANES)] for c in range(3)]
      for p, (ca, cb) in enumerate(((0, 1), (1, 2), (2, 0))):
        xf = (xc[ca] + 1.0) * 0.5 * (RES_ - 1)
        yf = (xc[cb] + 1.0) * 0.5 * (RES_ - 1)
        x0 = xf.astype(jnp.int32)
        y0 = yf.astype(jnp.int32)
        fx = xf - x0.astype(jnp.float32)
        fy = yf - y0.astype(jnp.float32)
        gx = 1.0 - fx
        gy = 1.0 - fy
        idxv[k * NPL + p, pl.ds(b, LANES)] = y0 * RES_ + x0 + (p * RES_ * RES_)
        for q, tw in enumerate((gy * gx, gy * fx, fy * gx, fy * fx)):
          wv[k * (4 * NPL) + p * 4 + q, pl.ds(b, LANES)] = tw
      return 0

    lax.fori_loop(0, KCH * NV, idx_body, 0)

    def acc_chunk(k):
      kb = k * B
      rbase = (k % NSLOT) * (NPL * B)

      def acc_body(i, _):
        b = i * LANES
        bi = iot + b
        for c in range(3):
          outv[pl.ds(sb + c * BP + kb + b, LANES)] = (
              xv[pl.ds(c * BP + kb + b, LANES)])
        ws = [wv[k * (4 * NPL) + t, pl.ds(b, LANES)] for t in range(4 * NPL)]
        rvs = [bi + rbase + p * B for p in range(NPL)]
        for ch in range(FDIM_):
          acc = None
          for p in range(NPL):
            for q in range(4):
              cols = jnp.full((LANES,), q * FDIM_ + ch, jnp.int32)
              v = plsc.load_gather(rows, [rvs[p], cols]) * ws[p * 4 + q]
              acc = v if acc is None else acc + v
          outv[pl.ds(sb + (3 + ch) * BP + kb + b, LANES)] = acc
        return 0

      lax.fori_loop(0, NV, acc_body, 0)

    def pipe_body(k, _):
      @pl.when(k >= NSLOT)
      def _():
        kk = k - NSLOT

        @pl.when(kk % 2 == 0)
        def _():
          gdrain(0, gsem0)

        @pl.when(kk % 2 == 1)
        def _():
          gdrain(1, gsem1)

        acc_chunk(kk)

      @pl.when(k < KCH)
      def _():
        @pl.when(k % 2 == 0)
        def _():
          gfire(k, 0, gsem0)

        @pl.when(k % 2 == 1)
        def _():
          gfire(k, 1, gsem1)

      return 0

    lax.fori_loop(0, KCH + NSLOT, pipe_body, 0)

    for r in range(OUTD):
      pltpu.async_copy(outv.at[pl.ds(sb + r * BP, BP)],
                       out_hbm.at[pl.ds(r * NPTS + pt0, BP)], osem)
    return 0

  lax.fori_loop(0, nb, batch_body, 0)
  odrain(0)
  odrain(0)


@jax.jit
def kernel(x, tgrid):
  tab = _build_quad_table(tgrid)
  mesh = plsc.VectorSubcoreMesh(core_axis_name="c", subcore_axis_name="s",
                                num_cores=NC, num_subcores=NS)
  run = pl.kernel(
      _tec_body,
      out_type=jax.ShapeDtypeStruct((OUTD * NPTS,), jnp.float32),
      mesh=mesh,
      compiler_params=pltpu.CompilerParams(needs_layout_passes=False,
                                           use_tc_tiling_on_sc=False),
      scratch_types=[
          pltpu.VMEM((3 * BP,), jnp.float32),
          pltpu.VMEM((KCH * NPL, B), jnp.int32),
          pltpu.VMEM((KCH * 4 * NPL, B), jnp.float32),
          pltpu.VMEM((NSLOT * NPL * B, QROW), jnp.float32),
          pltpu.VMEM((2 * OUTD * BP,), jnp.float32),
          pltpu.SemaphoreType.DMA,
          pltpu.SemaphoreType.DMA,
          pltpu.SemaphoreType.DMA,
          pltpu.SemaphoreType.DMA,
      ],
  )
  out_flat = run(x.T.reshape(NPTS * 3), tab)
  return out_flat.reshape(OUTD, NPTS).T


# speedup vs baseline: 1.4340x; 1.3783x over previous
import jax
import jax.numpy as jnp
from jax import lax
from jax.experimental import pallas as pl
from jax.experimental.pallas import tpu as pltpu
from jax.experimental.pallas import tpu_sc as plsc

RES_ = 512
FDIM_ = 32
NPTS = 1000000
OUTD = 3 + FDIM_
QROW = 4 * FDIM_

NC = 2
NS = 16
NW = NC * NS
LANES = 16

B = 80
NV = B // LANES
NPL = 3
KCH = 5
BP = KCH * B
NBATCH = NPTS // BP
NBW = NBATCH // NW
NBR = NBATCH % NW
NSLOT = 2


def _quad_body(a_ref, b_ref, t_ref):
  y = pl.program_id(1)
  a = a_ref[0, :, y % 8, :]
  b = b_ref[0, :, jnp.minimum(y + 1, RES_ - 1) % 8, :]
  eye = jnp.eye(FDIM_, dtype=jnp.float32)
  dn = (((0,), (0,)), ((), ()))
  at = lax.dot_general(a, eye, dn, precision=lax.Precision.HIGHEST)
  bt = lax.dot_general(b, eye, dn, precision=lax.Precision.HIGHEST)
  a1 = jnp.concatenate([at[1:], at[RES_ - 1:]], axis=0)
  b1 = jnp.concatenate([bt[1:], bt[RES_ - 1:]], axis=0)
  t_ref[...] = jnp.concatenate([at, a1, bt, b1], axis=1)


def _build_quad_table(tgrid):
  g4 = tgrid.reshape(3, FDIM_, RES_, RES_)
  line = pl.BlockSpec((1, FDIM_, 8, RES_), lambda p, y: (p, 0, y // 8, 0))
  line_n = pl.BlockSpec((1, FDIM_, 8, RES_),
                        lambda p, y: (p, 0, jnp.minimum(y + 1, RES_ - 1) // 8, 0))
  return pl.pallas_call(
      _quad_body,
      grid=(3, RES_),
      in_specs=[line, line_n],
      out_specs=pl.BlockSpec((RES_, QROW), lambda p, y: (p * RES_ + y, 0)),
      out_shape=jax.ShapeDtypeStruct((3 * RES_ * RES_, QROW), jnp.float32),
  )(g4, g4)


def _tec_body(x_hbm, tab_hbm, out_hbm, xv, idxv, wv, rows, outv,
              gsem0, gsem1, osem, xsem):
  cid = lax.axis_index("c")
  sid = lax.axis_index("s")
  wid = sid * NC + cid
  nb = NBW + jnp.where(wid < NBR, 1, 0)
  b0 = wid * NBW + jnp.minimum(wid, NBR)
  iot = lax.iota(jnp.int32, LANES)

  def gfire(k, slot, sem):
    for p in range(NPL):
      pltpu.async_copy(tab_hbm.at[idxv.at[k * NPL + p]],
                       rows.at[pl.ds((slot * NPL + p) * B, B), :], sem)

  def gdrain(slot, sem):
    for p in range(NPL):
      pltpu.make_async_copy(tab_hbm.at[idxv.at[p]],
                            rows.at[pl.ds((slot * NPL + p) * B, B), :],
                            sem).wait()

  def odrain(pt0):
    for r in range(OUTD):
      pltpu.make_async_copy(outv.at[pl.ds(r * BP, BP)],
                            out_hbm.at[pl.ds(r * PSTRIDE + pt0, BP)], osem).wait()

  def batch_body(g, _):
    pt0 = (b0 + g) * BP
    sb = (g % 2) * (OUTD * BP)

    @pl.when(g >= 2)
    def _():
      odrain(pt0)

    for c in range(3):
      pltpu.async_copy(x_hbm.at[pl.ds(c * NPTS + pt0, BP)],
                       xv.at[pl.ds(c * BP, BP)], xsem)
    for c in range(3):
      pltpu.make_async_copy(x_hbm.at[pl.ds(c * NPTS + pt0, BP)],
                            xv.at[pl.ds(c * BP, BP)], xsem).wait()

    def idx_body(i, _):
      k = i // NV
      b = (i % NV) * LANES
      xc = [xv[pl.ds(c * BP + i * LANES, LANES)] for c in range(3)]
      for p, (ca, cb) in enumerate(((0, 1), (1, 2), (2, 0))):
        xf = (xc[ca] + 1.0) * 0.5 * (RES_ - 1)
        yf = (xc[cb] + 1.0) * 0.5 * (RES_ - 1)
        x0 = xf.astype(jnp.int32)
        y0 = yf.astype(jnp.int32)
        fx = xf - x0.astype(jnp.float32)
        fy = yf - y0.astype(jnp.float32)
        gx = 1.0 - fx
        gy = 1.0 - fy
        idxv[k * NPL + p, pl.ds(b, LANES)] = y0 * RES_ + x0 + (p * RES_ * RES_)
        for q, tw in enumerate((gy * gx, gy * fx, fy * gx, fy * fx)):
          wv[k * (4 * NPL) + p * 4 + q, pl.ds(b, LANES)] = tw
      return 0

    lax.fori_loop(0, KCH * NV, idx_body, 0)

    def acc_chunk(k):
      kb = k * B
      rbase = (k % NSLOT) * (NPL * B)

      def acc_body(i, _):
        b = i * LANES
        bi = iot + b
        for c in range(3):
          outv[pl.ds(sb + c * BP + kb + b, LANES)] = (
              xv[pl.ds(c * BP + kb + b, LANES)])
        ws = [wv[k * (4 * NPL) + t, pl.ds(b, LANES)] for t in range(4 * NPL)]
        rvs = [bi + rbase + p * B for p in range(NPL)]
        for ch in range(FDIM_):
          acc = None
          for p in range(NPL):
            for q in range(4):
              cols = jnp.full((LANES,), q * FDIM_ + ch, jnp.int32)
              v = plsc.load_gather(rows, [rvs[p], cols]) * ws[p * 4 + q]
              acc = v if acc is None else acc + v
          outv[pl.ds(sb + (3 + ch) * BP + kb + b, LANES)] = acc
        return 0

      lax.fori_loop(0, NV, acc_body, 0)

    def pipe_body(k, _):
      @pl.when(k >= NSLOT)
      def _():
        kk = k - NSLOT

        @pl.when(kk % 2 == 0)
        def _():
          gdrain(0, gsem0)

        @pl.when(kk % 2 == 1)
        def _():
          gdrain(1, gsem1)

        acc_chunk(kk)

      @pl.when(k < KCH)
      def _():
        @pl.when(k % 2 == 0)
        def _():
          gfire(k, 0, gsem0)

        @pl.when(k % 2 == 1)
        def _():
          gfire(k, 1, gsem1)

      return 0

    lax.fori_loop(0, KCH + NSLOT, pipe_body, 0)

    for r in range(OUTD):
      pltpu.async_copy(outv.at[pl.ds(sb + r * BP, BP)],
                       out_hbm.at[pl.ds(r * PSTRIDE + pt0, BP)], osem)
    return 0

  lax.fori_loop(0, nb, batch_body, 0)
  odrain(0)
  odrain(0)


PTB = 128000
PSTRIDE = 8 * PTB


def _untile_body(*refs):
  ins, t_ref = refs[:8], refs[8]
  t_ref[...] = jnp.stack([r[...] for r in ins], axis=0)


def _untile(out_flat):
  in_specs = [
      pl.BlockSpec((PTB,),
                   lambda g, pb, r=r: (jnp.minimum(8 * g + r, OUTD - 1)
                                       * 8 + pb,))
      for r in range(8)
  ]
  return pl.pallas_call(
      _untile_body,
      grid=(5, 8),
      in_specs=in_specs,
      out_specs=pl.BlockSpec((8, PTB), lambda g, pb: (g, pb)),
      out_shape=jax.ShapeDtypeStruct((OUTD, NPTS), jnp.float32),
  )(*([out_flat] * 8))


@jax.jit
def kernel(x, tgrid):
  tab = _build_quad_table(tgrid)
  mesh = plsc.VectorSubcoreMesh(core_axis_name="c", subcore_axis_name="s",
                                num_cores=NC, num_subcores=NS)
  run = pl.kernel(
      _tec_body,
      out_type=jax.ShapeDtypeStruct((OUTD * PSTRIDE,), jnp.float32),
      mesh=mesh,
      compiler_params=pltpu.CompilerParams(needs_layout_passes=False,
                                           use_tc_tiling_on_sc=False),
      scratch_types=[
          pltpu.VMEM((3 * BP,), jnp.float32),
          pltpu.VMEM((KCH * NPL, B), jnp.int32),
          pltpu.VMEM((KCH * 4 * NPL, B), jnp.float32),
          pltpu.VMEM((NSLOT * NPL * B, QROW), jnp.float32),
          pltpu.VMEM((2 * OUTD * BP,), jnp.float32),
          pltpu.SemaphoreType.DMA,
          pltpu.SemaphoreType.DMA,
          pltpu.SemaphoreType.DMA,
          pltpu.SemaphoreType.DMA,
      ],
  )
  out_flat = run(x.T.reshape(NPTS * 3), tab)
  return _untile(out_flat).T


# speedup vs baseline: 1.5868x; 1.1066x over previous
import jax
import jax.numpy as jnp
from jax import lax
from jax.experimental import pallas as pl
from jax.experimental.pallas import tpu as pltpu
from jax.experimental.pallas import tpu_sc as plsc

RES_ = 512
FDIM_ = 32
NPTS = 1000000
OUTD = 3 + FDIM_
QROW = 4 * FDIM_

NC = 2
NS = 16
NW = NC * NS
LANES = 16

B = 80
NV = B // LANES
NPL = 3
KCH = 5
BP = KCH * B
NBATCH = NPTS // BP
NBW = NBATCH // NW
NBR = NBATCH % NW
NSLOT = 2


def _quad_body(a_ref, b_ref, t_ref):
  g = pl.program_id(1)
  eye = jnp.eye(FDIM_, dtype=jnp.float32)
  dn = (((0,), (0,)), ((), ()))
  a2 = a_ref[0].reshape(FDIM_, 8 * RES_)
  tl = lax.dot_general(a2, eye, dn, precision=lax.Precision.HIGHEST)
  t3 = tl.reshape(8, RES_, FDIM_)
  b2 = b_ref[0, :, 0, :]
  tb0 = lax.dot_general(b2, eye, dn, precision=lax.Precision.HIGHEST)
  tb0 = jnp.where(g == RES_ // 8 - 1, t3[7], tb0)
  tn3 = jnp.concatenate([t3[1:], tb0[None]], axis=0)
  t3s = jnp.concatenate([t3[:, 1:, :], t3[:, RES_ - 1:, :]], axis=1)
  tn3s = jnp.concatenate([tn3[:, 1:, :], tn3[:, RES_ - 1:, :]], axis=1)
  quad = jnp.concatenate([t3, t3s, tn3, tn3s], axis=2)
  t_ref[...] = quad.reshape(8 * RES_, QROW)


def _build_quad_table(tgrid):
  g4 = tgrid.reshape(3, FDIM_, RES_, RES_)
  lines = pl.BlockSpec((1, FDIM_, 8, RES_), lambda p, g: (p, 0, g, 0))
  lines_n = pl.BlockSpec(
      (1, FDIM_, 8, RES_),
      lambda p, g: (p, 0, jnp.minimum(g + 1, RES_ // 8 - 1), 0))
  return pl.pallas_call(
      _quad_body,
      grid=(3, RES_ // 8),
      in_specs=[lines, lines_n],
      out_specs=pl.BlockSpec((8 * RES_, QROW), lambda p, g: (p * (RES_ // 8) + g, 0)),
      out_shape=jax.ShapeDtypeStruct((3 * RES_ * RES_, QROW), jnp.float32),
  )(g4, g4)


def _tec_body(x_hbm, tab_hbm, out_hbm, xv, idxv, wv, rows, outv,
              gsem0, gsem1, osem, xsem):
  cid = lax.axis_index("c")
  sid = lax.axis_index("s")
  wid = sid * NC + cid
  nb = NBW + jnp.where(wid < NBR, 1, 0)
  b0 = wid * NBW + jnp.minimum(wid, NBR)
  iot = lax.iota(jnp.int32, LANES)

  def gfire(k, slot, sem):
    for p in range(NPL):
      pltpu.async_copy(tab_hbm.at[idxv.at[k * NPL + p]],
                       rows.at[pl.ds((slot * NPL + p) * B, B), :], sem)

  def gdrain(slot, sem):
    for p in range(NPL):
      pltpu.make_async_copy(tab_hbm.at[idxv.at[p]],
                            rows.at[pl.ds((slot * NPL + p) * B, B), :],
                            sem).wait()

  def odrain(pt0):
    for r in range(OUTD):
      pltpu.make_async_copy(outv.at[pl.ds(r * BP, BP)],
                            out_hbm.at[pl.ds(r * PSTRIDE + pt0, BP)], osem).wait()

  def batch_body(g, _):
    pt0 = (b0 + g) * BP
    sb = (g % 2) * (OUTD * BP)

    @pl.when(g >= 2)
    def _():
      odrain(pt0)

    for c in range(3):
      pltpu.async_copy(x_hbm.at[pl.ds(c * NPTS + pt0, BP)],
                       xv.at[pl.ds(c * BP, BP)], xsem)
    for c in range(3):
      pltpu.make_async_copy(x_hbm.at[pl.ds(c * NPTS + pt0, BP)],
                            xv.at[pl.ds(c * BP, BP)], xsem).wait()

    def idx_body(i, _):
      k = i // NV
      b = (i % NV) * LANES
      xc = [xv[pl.ds(c * BP + i * LANES, LANES)] for c in range(3)]
      for p, (ca, cb) in enumerate(((0, 1), (1, 2), (2, 0))):
        xf = (xc[ca] + 1.0) * 0.5 * (RES_ - 1)
        yf = (xc[cb] + 1.0) * 0.5 * (RES_ - 1)
        x0 = xf.astype(jnp.int32)
        y0 = yf.astype(jnp.int32)
        fx = xf - x0.astype(jnp.float32)
        fy = yf - y0.astype(jnp.float32)
        gx = 1.0 - fx
        gy = 1.0 - fy
        idxv[k * NPL + p, pl.ds(b, LANES)] = y0 * RES_ + x0 + (p * RES_ * RES_)
        for q, tw in enumerate((gy * gx, gy * fx, fy * gx, fy * fx)):
          wv[k * (4 * NPL) + p * 4 + q, pl.ds(b, LANES)] = tw
      return 0

    lax.fori_loop(0, KCH * NV, idx_body, 0)

    def acc_chunk(k):
      kb = k * B
      rbase = (k % NSLOT) * (NPL * B)

      def acc_body(i, _):
        b = i * LANES
        bi = iot + b
        for c in range(3):
          outv[pl.ds(sb + c * BP + kb + b, LANES)] = (
              xv[pl.ds(c * BP + kb + b, LANES)])
        ws = [wv[k * (4 * NPL) + t, pl.ds(b, LANES)] for t in range(4 * NPL)]
        rvs = [bi + rbase + p * B for p in range(NPL)]
        for ch in range(FDIM_):
          acc = None
          for p in range(NPL):
            for q in range(4):
              cols = jnp.full((LANES,), q * FDIM_ + ch, jnp.int32)
              v = plsc.load_gather(rows, [rvs[p], cols]) * ws[p * 4 + q]
              acc = v if acc is None else acc + v
          outv[pl.ds(sb + (3 + ch) * BP + kb + b, LANES)] = acc
        return 0

      lax.fori_loop(0, NV, acc_body, 0)

    def pipe_body(k, _):
      @pl.when(k >= NSLOT)
      def _():
        kk = k - NSLOT

        @pl.when(kk % 2 == 0)
        def _():
          gdrain(0, gsem0)

        @pl.when(kk % 2 == 1)
        def _():
          gdrain(1, gsem1)

        acc_chunk(kk)

      @pl.when(k < KCH)
      def _():
        @pl.when(k % 2 == 0)
        def _():
          gfire(k, 0, gsem0)

        @pl.when(k % 2 == 1)
        def _():
          gfire(k, 1, gsem1)

      return 0

    lax.fori_loop(0, KCH + NSLOT, pipe_body, 0)

    for r in range(OUTD):
      pltpu.async_copy(outv.at[pl.ds(sb + r * BP, BP)],
                       out_hbm.at[pl.ds(r * PSTRIDE + pt0, BP)], osem)
    return 0

  lax.fori_loop(0, nb, batch_body, 0)
  odrain(0)
  odrain(0)


PTB = 128000
PSTRIDE = 8 * PTB


def _untile_body(*refs):
  ins, t_ref = refs[:8], refs[8]
  t_ref[...] = jnp.stack([r[...] for r in ins], axis=0)


def _untile(out_flat):
  in_specs = [
      pl.BlockSpec((PTB,),
                   lambda g, pb, r=r: (jnp.minimum(8 * g + r, OUTD - 1)
                                       * 8 + pb,))
      for r in range(8)
  ]
  return pl.pallas_call(
      _untile_body,
      grid=(5, 8),
      in_specs=in_specs,
      out_specs=pl.BlockSpec((8, PTB), lambda g, pb: (g, pb)),
      out_shape=jax.ShapeDtypeStruct((OUTD, NPTS), jnp.float32),
  )(*([out_flat] * 8))


@jax.jit
def kernel(x, tgrid):
  tab = _build_quad_table(tgrid)
  mesh = plsc.VectorSubcoreMesh(core_axis_name="c", subcore_axis_name="s",
                                num_cores=NC, num_subcores=NS)
  run = pl.kernel(
      _tec_body,
      out_type=jax.ShapeDtypeStruct((OUTD * PSTRIDE,), jnp.float32),
      mesh=mesh,
      compiler_params=pltpu.CompilerParams(needs_layout_passes=False,
                                           use_tc_tiling_on_sc=False),
      scratch_types=[
          pltpu.VMEM((3 * BP,), jnp.float32),
          pltpu.VMEM((KCH * NPL, B), jnp.int32),
          pltpu.VMEM((KCH * 4 * NPL, B), jnp.float32),
          pltpu.VMEM((NSLOT * NPL * B, QROW), jnp.float32),
          pltpu.VMEM((2 * OUTD * BP,), jnp.float32),
          pltpu.SemaphoreType.DMA,
          pltpu.SemaphoreType.DMA,
          pltpu.SemaphoreType.DMA,
          pltpu.SemaphoreType.DMA,
      ],
  )
  out_flat = run(x.T.reshape(NPTS * 3), tab)
  return _untile(out_flat).T
